# Initial kernel scaffold; baseline (speedup 1.0000x reference)
import jax
import jax.numpy as jnp
from jax.experimental import pallas as pl


def kernel(edges, nodes):
    raise NotImplementedError("write your pallas kernel here")


# speedup vs baseline: 8.4157x; 8.4157x over previous
import functools

import jax
import jax.numpy as jnp
from jax import lax
from jax.experimental import pallas as pl
from jax.experimental.pallas import tpu as pltpu
from jax.experimental.pallas import tpu_sc as plsc

_N_EDGES = 160000
_N = 2 * _N_EDGES
_NW = 16
_S = _N // _NW
_NB = 10240
_BR = _NB // _NW
_NG = _S // 16
_CHUNK = 128
_NCH = _S // _CHUNK
_PAD = _CHUNK - (_S - _NCH * _CHUNK)
_OUT_S = _N + _NW * _PAD


def _body(src_hbm, dst_hbm, out_hbm, keys_v, vals_v, hist_v, block_v, loc_v,
          acc_v, tots_v, posrow_v, carry_s, hist_all_s, totals_s, out_s):
  wid = lax.axis_index("s")
  zeros = jnp.zeros((16,), jnp.int32)

  off = (wid % 8) * _S

  @pl.when(wid < 8)
  def _():
    pltpu.sync_copy(src_hbm.at[pl.ds(off, _S)], keys_v)
    pltpu.sync_copy(dst_hbm.at[pl.ds(off, _S)], vals_v.at[pl.ds(0, _S)])

  @pl.when(wid >= 8)
  def _():
    pltpu.sync_copy(dst_hbm.at[pl.ds(off, _S)], keys_v)
    pltpu.sync_copy(src_hbm.at[pl.ds(off, _S)], vals_v.at[pl.ds(0, _S)])

  @pl.loop(0, _NB // 16)
  def _(i):
    hist_v[pl.ds(i * 16, 16)] = zeros

  @pl.loop(0, _NG)
  def _(i):
    k = keys_v[pl.ds(i * 16, 16)]
    run, last = plsc.scan_count(k)
    plsc.addupdate_scatter(hist_v, [k], run, mask=last)

  pltpu.sync_copy(hist_v, hist_all_s.at[pl.ds(wid * _NB, _NB)])
  plsc.subcore_barrier()

  for w2 in range(_NW):
    pltpu.sync_copy(hist_all_s.at[pl.ds(w2 * _NB + wid * _BR, _BR)],
                    block_v.at[pl.ds(w2 * _BR, _BR)])

  @pl.loop(0, _BR // 16)
  def _(g):
    acc_v[pl.ds(g * 16, 16)] = zeros

  for w2 in range(_NW):

    @pl.loop(0, _BR // 16)
    def _(g, w2=w2):
      a = acc_v[pl.ds(g * 16, 16)]
      h = block_v[pl.ds(w2 * _BR + g * 16, 16)]
      block_v[pl.ds(w2 * _BR + g * 16, 16)] = a
      acc_v[pl.ds(g * 16, 16)] = a + h

  carry_s[0] = 0

  @pl.loop(0, _BR // 16)
  def _(g):
    v = acc_v[pl.ds(g * 16, 16)]
    c = plsc.cumsum(v)
    cin = carry_s[0]
    loc_v[pl.ds(g * 16, 16)] = c - v + cin
    carry_s[0] = cin + jnp.sum(v)

  tots_v[...] = jnp.full((16,), carry_s[0], jnp.int32)
  pltpu.sync_copy(tots_v, totals_s.at[pl.ds(wid * 16, 16)])
  plsc.subcore_barrier()

  pvec = zeros
  for w2 in range(_NW):
    pltpu.sync_copy(totals_s.at[pl.ds(w2 * 16, 16)], tots_v)
    gate = jnp.where(w2 < wid, 1, 0).astype(jnp.int32)
    pvec = pvec + tots_v[...] * gate

  for w2 in range(_NW):

    @pl.loop(0, _BR // 16)
    def _(g, w2=w2, pvec=pvec):
      o = w2 * _BR + g * 16
      block_v[pl.ds(o, 16)] = (block_v[pl.ds(o, 16)] +
                               loc_v[pl.ds(g * 16, 16)] + pvec)

    pltpu.sync_copy(block_v.at[pl.ds(w2 * _BR, _BR)],
                    hist_all_s.at[pl.ds(w2 * _NB + wid * _BR, _BR)])
  plsc.subcore_barrier()

  pltpu.sync_copy(hist_all_s.at[pl.ds(wid * _NB, _NB)], hist_v)

  @pl.loop(0, _NCH)
  def _(c):
    for j in range(_CHUNK // 16):
      k = keys_v[pl.ds(c * _CHUNK + j * 16, 16)]
      run, last = plsc.scan_count(k)
      cur = plsc.load_gather(hist_v, [k])
      posrow_v[pl.ds(j * 16, 16)] = cur + run - 1
      plsc.addupdate_scatter(hist_v, [k], run, mask=last)
    pltpu.sync_copy(vals_v.at[pl.ds(c * _CHUNK, _CHUNK)], out_s.at[posrow_v])

  iot = lax.iota(jnp.int32, 16)
  for j in range(2):
    k = keys_v[pl.ds(_NCH * _CHUNK + j * 16, 16)]
    run, last = plsc.scan_count(k)
    cur = plsc.load_gather(hist_v, [k])
    posrow_v[pl.ds(j * 16, 16)] = cur + run - 1
    plsc.addupdate_scatter(hist_v, [k], run, mask=last)
  for j in range(2, _CHUNK // 16):
    posrow_v[pl.ds(j * 16, 16)] = _N + wid * _PAD + (j - 2) * 16 + iot
  pltpu.sync_copy(vals_v.at[pl.ds(_NCH * _CHUNK, _CHUNK)], out_s.at[posrow_v])
  plsc.subcore_barrier()

  pltpu.sync_copy(out_s.at[pl.ds(wid * _S, _S)], keys_v)
  pltpu.sync_copy(keys_v, out_hbm.at[pl.ds(wid * _S, _S)])


_sort = pl.kernel(
    _body,
    out_type=jax.ShapeDtypeStruct((_N,), jnp.int32),
    mesh=plsc.VectorSubcoreMesh(
        core_axis_name="c", subcore_axis_name="s", num_cores=1),
    compiler_params=pltpu.CompilerParams(needs_layout_passes=False),
    scratch_types=[
        pltpu.VMEM((_S,), jnp.int32),
        pltpu.VMEM(((_NCH + 1) * _CHUNK,), jnp.int32),
        pltpu.VMEM((_NB,), jnp.int32),
        pltpu.VMEM((_NB,), jnp.int32),
        pltpu.VMEM((_BR,), jnp.int32),
        pltpu.VMEM((_BR,), jnp.int32),
        pltpu.VMEM((16,), jnp.int32),
        pltpu.VMEM((_CHUNK,), jnp.int32),
        pltpu.SMEM((1,), jnp.int32),
        pltpu.VMEM_SHARED((_NW * _NB,), jnp.int32),
        pltpu.VMEM_SHARED((_NW * 16,), jnp.int32),
        pltpu.VMEM_SHARED((_OUT_S,), jnp.int32),
    ],
)


@jax.jit
def kernel(edges, nodes):
  e = edges.astype(jnp.int32)
  flat = _sort(e[:, 0], e[:, 1])
  return flat.reshape(nodes.shape[0], -1)


# speedup vs baseline: 9.1944x; 1.0925x over previous
import functools

import jax
import jax.numpy as jnp
from jax import lax
from jax.experimental import pallas as pl
from jax.experimental.pallas import tpu as pltpu
from jax.experimental.pallas import tpu_sc as plsc

_N_EDGES = 160000
_N = 2 * _N_EDGES
_NW = 16
_NV = 2 * _NW
_S = _N // _NW
_SV = _N // _NV
_NB = 10240
_BR = _NB // _NW
_CHUNK = 128
_NCH = _SV // _CHUNK
_PAD = _CHUNK - (_SV - _NCH * _CHUNK)
_OUT_S = _N + _NV * _PAD


def _body(src_hbm, dst_hbm, out_hbm, keys_v, vals_v, hist_a, hist_b, block_v,
          loc_v, acc_v, tots_v, posrow_a, posrow_b, carry_s, hist_all_s,
          totals_s, out_s):
  wid = lax.axis_index("s")
  zeros = jnp.zeros((16,), jnp.int32)

  off = (wid % 8) * _S

  @pl.when(wid < 8)
  def _():
    pltpu.sync_copy(src_hbm.at[pl.ds(off, _S)], keys_v)
    pltpu.sync_copy(dst_hbm.at[pl.ds(off, _S)], vals_v.at[pl.ds(0, _S)])

  @pl.when(wid >= 8)
  def _():
    pltpu.sync_copy(dst_hbm.at[pl.ds(off, _S)], keys_v)
    pltpu.sync_copy(src_hbm.at[pl.ds(off, _S)], vals_v.at[pl.ds(0, _S)])

  @pl.loop(0, _NB // 16)
  def _(i):
    hist_a[pl.ds(i * 16, 16)] = zeros
    hist_b[pl.ds(i * 16, 16)] = zeros

  @pl.loop(0, _SV // 16)
  def _(i):
    ka = keys_v[pl.ds(i * 16, 16)]
    kb = keys_v[pl.ds(_SV + i * 16, 16)]
    run_a, last_a = plsc.scan_count(ka)
    run_b, last_b = plsc.scan_count(kb)
    plsc.addupdate_scatter(hist_a, [ka], run_a, mask=last_a)
    plsc.addupdate_scatter(hist_b, [kb], run_b, mask=last_b)

  pltpu.sync_copy(hist_a, hist_all_s.at[pl.ds((2 * wid) * _NB, _NB)])
  pltpu.sync_copy(hist_b, hist_all_s.at[pl.ds((2 * wid + 1) * _NB, _NB)])
  plsc.subcore_barrier()

  for v2 in range(_NV):
    pltpu.sync_copy(hist_all_s.at[pl.ds(v2 * _NB + wid * _BR, _BR)],
                    block_v.at[pl.ds(v2 * _BR, _BR)])

  @pl.loop(0, _BR // 16)
  def _(g):
    acc_v[pl.ds(g * 16, 16)] = zeros

  for v2 in range(_NV):

    @pl.loop(0, _BR // 16)
    def _(g, v2=v2):
      a = acc_v[pl.ds(g * 16, 16)]
      h = block_v[pl.ds(v2 * _BR + g * 16, 16)]
      block_v[pl.ds(v2 * _BR + g * 16, 16)] = a
      acc_v[pl.ds(g * 16, 16)] = a + h

  carry_s[0] = 0

  @pl.loop(0, _BR // 16)
  def _(g):
    v = acc_v[pl.ds(g * 16, 16)]
    c = plsc.cumsum(v)
    cin = carry_s[0]
    loc_v[pl.ds(g * 16, 16)] = c - v + cin
    carry_s[0] = cin + jnp.sum(v)

  tots_v[...] = jnp.full((16,), carry_s[0], jnp.int32)
  pltpu.sync_copy(tots_v, totals_s.at[pl.ds(wid * 16, 16)])
  plsc.subcore_barrier()

  pvec = zeros
  for w2 in range(_NW):
    pltpu.sync_copy(totals_s.at[pl.ds(w2 * 16, 16)], tots_v)
    gate = jnp.where(w2 < wid, 1, 0).astype(jnp.int32)
    pvec = pvec + tots_v[...] * gate

  for v2 in range(_NV):

    @pl.loop(0, _BR // 16)
    def _(g, v2=v2, pvec=pvec):
      o = v2 * _BR + g * 16
      block_v[pl.ds(o, 16)] = (block_v[pl.ds(o, 16)] +
                               loc_v[pl.ds(g * 16, 16)] + pvec)

    pltpu.sync_copy(block_v.at[pl.ds(v2 * _BR, _BR)],
                    hist_all_s.at[pl.ds(v2 * _NB + wid * _BR, _BR)])
  plsc.subcore_barrier()

  pltpu.sync_copy(hist_all_s.at[pl.ds((2 * wid) * _NB, _NB)], hist_a)
  pltpu.sync_copy(hist_all_s.at[pl.ds((2 * wid + 1) * _NB, _NB)], hist_b)

  @pl.loop(0, _NCH)
  def _(c):
    for j in range(_CHUNK // 16):
      ka = keys_v[pl.ds(c * _CHUNK + j * 16, 16)]
      kb = keys_v[pl.ds(_SV + c * _CHUNK + j * 16, 16)]
      run_a, last_a = plsc.scan_count(ka)
      run_b, last_b = plsc.scan_count(kb)
      cur_a = plsc.load_gather(hist_a, [ka])
      cur_b = plsc.load_gather(hist_b, [kb])
      posrow_a[pl.ds(j * 16, 16)] = cur_a + run_a - 1
      posrow_b[pl.ds(j * 16, 16)] = cur_b + run_b - 1
      plsc.addupdate_scatter(hist_a, [ka], run_a, mask=last_a)
      plsc.addupdate_scatter(hist_b, [kb], run_b, mask=last_b)
    pltpu.sync_copy(vals_v.at[pl.ds(c * _CHUNK, _CHUNK)], out_s.at[posrow_a])
    pltpu.sync_copy(vals_v.at[pl.ds(_SV + c * _CHUNK, _CHUNK)],
                    out_s.at[posrow_b])

  iot = lax.iota(jnp.int32, 16)
  ka = keys_v[pl.ds(_NCH * _CHUNK, 16)]
  kb = keys_v[pl.ds(_SV + _NCH * _CHUNK, 16)]
  run_a, last_a = plsc.scan_count(ka)
  run_b, last_b = plsc.scan_count(kb)
  cur_a = plsc.load_gather(hist_a, [ka])
  cur_b = plsc.load_gather(hist_b, [kb])
  posrow_a[pl.ds(0, 16)] = cur_a + run_a - 1
  posrow_b[pl.ds(0, 16)] = cur_b + run_b - 1
  for j in range(1, _CHUNK // 16):
    posrow_a[pl.ds(j * 16, 16)] = _N + (2 * wid) * _PAD + (j - 1) * 16 + iot
    posrow_b[pl.ds(j * 16, 16)] = (_N + (2 * wid + 1) * _PAD +
                                   (j - 1) * 16 + iot)
  pltpu.sync_copy(vals_v.at[pl.ds(_NCH * _CHUNK, _CHUNK)], out_s.at[posrow_a])
  pltpu.sync_copy(vals_v.at[pl.ds(_SV + _NCH * _CHUNK, _CHUNK)],
                  out_s.at[posrow_b])
  plsc.subcore_barrier()

  pltpu.sync_copy(out_s.at[pl.ds(wid * _S, _S)], keys_v)
  pltpu.sync_copy(keys_v, out_hbm.at[pl.ds(wid * _S, _S)])


_sort = pl.kernel(
    _body,
    out_type=jax.ShapeDtypeStruct((_N,), jnp.int32),
    mesh=plsc.VectorSubcoreMesh(
        core_axis_name="c", subcore_axis_name="s", num_cores=1),
    compiler_params=pltpu.CompilerParams(needs_layout_passes=False),
    scratch_types=[
        pltpu.VMEM((_S,), jnp.int32),
        pltpu.VMEM((_S + _PAD,), jnp.int32),
        pltpu.VMEM((_NB,), jnp.int32),
        pltpu.VMEM((_NB,), jnp.int32),
        pltpu.VMEM((_NV * _BR,), jnp.int32),
        pltpu.VMEM((_BR,), jnp.int32),
        pltpu.VMEM((_BR,), jnp.int32),
        pltpu.VMEM((16,), jnp.int32),
        pltpu.VMEM((_CHUNK,), jnp.int32),
        pltpu.VMEM((_CHUNK,), jnp.int32),
        pltpu.SMEM((1,), jnp.int32),
        pltpu.VMEM_SHARED((_NV * _NB,), jnp.int32),
        pltpu.VMEM_SHARED((_NW * 16,), jnp.int32),
        pltpu.VMEM_SHARED((_OUT_S,), jnp.int32),
    ],
)


@jax.jit
def kernel(edges, nodes):
  e = edges.astype(jnp.int32)
  flat = _sort(e[:, 0], e[:, 1])
  return flat.reshape(nodes.shape[0], -1)


# speedup vs baseline: 12.0730x; 1.3131x over previous
import functools

import jax
import jax.numpy as jnp
from jax import lax
from jax.experimental import pallas as pl
from jax.experimental.pallas import tpu as pltpu
from jax.experimental.pallas import tpu_sc as plsc

_N_EDGES = 160000
_N = 2 * _N_EDGES
_NW = 16
_NV = 2 * _NW
_S = _N // _NW
_SV = _N // _NV
_NB = 10240
_BR = _NB // _NW
_CHUNK = 128
_NCH = _SV // _CHUNK
_PAD = _CHUNK - (_SV - _NCH * _CHUNK)
_OUT_S = _N + _NV * _PAD


def _body(src_hbm, dst_hbm, out_hbm, keys_v, vals_v, hist_a, hist_b,
          block_v, loc_v, acc_v, tots_v, pos_a0, pos_a1, pos_b0, pos_b1,
          carry_s, hist_all_s, totals_s, out_s, sem_k, sem_v, sem_h,
          sem_a0, sem_a1, sem_b0, sem_b1):
  wid = lax.axis_index("s")
  zeros = jnp.zeros((16,), jnp.int32)

  off = (wid % 8) * _S

  @pl.when(wid < 8)
  def _():
    pltpu.async_copy(src_hbm.at[pl.ds(off, _S)], keys_v, sem_k)
    pltpu.async_copy(dst_hbm.at[pl.ds(off, _S)], vals_v.at[pl.ds(0, _S)],
                     sem_v)

  @pl.when(wid >= 8)
  def _():
    pltpu.async_copy(dst_hbm.at[pl.ds(off, _S)], keys_v, sem_k)
    pltpu.async_copy(src_hbm.at[pl.ds(off, _S)], vals_v.at[pl.ds(0, _S)],
                     sem_v)

  @pl.loop(0, _NB // 16)
  def _(i):
    hist_a[pl.ds(i * 16, 16)] = zeros
    hist_b[pl.ds(i * 16, 16)] = zeros

  pltpu.make_async_copy(src_hbm.at[pl.ds(off, _S)], keys_v, sem_k).wait()

  @pl.loop(0, _SV // 16)
  def _(i):
    ka = keys_v[pl.ds(i * 16, 16)]
    kb = keys_v[pl.ds(_SV + i * 16, 16)]
    run_a, last_a = plsc.scan_count(ka)
    run_b, last_b = plsc.scan_count(kb)
    cur_a = plsc.load_gather(hist_a, [ka])
    cur_b = plsc.load_gather(hist_b, [kb])
    keys_v[pl.ds(i * 16, 16)] = ka + ((cur_a + run_a - 1) << 14)
    keys_v[pl.ds(_SV + i * 16, 16)] = kb + ((cur_b + run_b - 1) << 14)
    plsc.addupdate_scatter(hist_a, [ka], run_a, mask=last_a)
    plsc.addupdate_scatter(hist_b, [kb], run_b, mask=last_b)

  pltpu.async_copy(hist_a, hist_all_s.at[pl.ds((2 * wid) * _NB, _NB)], sem_h)
  pltpu.async_copy(hist_b, hist_all_s.at[pl.ds((2 * wid + 1) * _NB, _NB)],
                   sem_h)
  pltpu.make_async_copy(hist_a, hist_all_s.at[pl.ds((2 * wid) * _NB, _NB)],
                        sem_h).wait()
  pltpu.make_async_copy(hist_b, hist_all_s.at[pl.ds((2 * wid) * _NB, _NB)],
                        sem_h).wait()
  plsc.subcore_barrier()

  for v2 in range(_NV):
    pltpu.async_copy(hist_all_s.at[pl.ds(v2 * _NB + wid * _BR, _BR)],
                     block_v.at[pl.ds(v2 * _BR, _BR)], sem_h)
  for v2 in range(_NV):
    pltpu.make_async_copy(hist_all_s.at[pl.ds(v2 * _NB + wid * _BR, _BR)],
                          block_v.at[pl.ds(v2 * _BR, _BR)], sem_h).wait()

  @pl.loop(0, _BR // 16)
  def _(g):
    acc_v[pl.ds(g * 16, 16)] = zeros

  for v2 in range(_NV):

    @pl.loop(0, _BR // 16)
    def _(g, v2=v2):
      a = acc_v[pl.ds(g * 16, 16)]
      h = block_v[pl.ds(v2 * _BR + g * 16, 16)]
      block_v[pl.ds(v2 * _BR + g * 16, 16)] = a
      acc_v[pl.ds(g * 16, 16)] = a + h

  carry_s[0] = 0

  @pl.loop(0, _BR // 16)
  def _(g):
    v = acc_v[pl.ds(g * 16, 16)]
    c = plsc.cumsum(v)
    cin = carry_s[0]
    loc_v[pl.ds(g * 16, 16)] = c - v + cin
    carry_s[0] = cin + jnp.sum(v)

  tots_v[...] = jnp.full((16,), carry_s[0], jnp.int32)
  pltpu.sync_copy(tots_v, totals_s.at[pl.ds(wid * 16, 16)])
  plsc.subcore_barrier()

  pvec = zeros
  for w2 in range(_NW):
    pltpu.sync_copy(totals_s.at[pl.ds(w2 * 16, 16)], tots_v)
    gate = jnp.where(w2 < wid, 1, 0).astype(jnp.int32)
    pvec = pvec + tots_v[...] * gate

  for v2 in range(_NV):

    @pl.loop(0, _BR // 16)
    def _(g, v2=v2, pvec=pvec):
      o = v2 * _BR + g * 16
      block_v[pl.ds(o, 16)] = (block_v[pl.ds(o, 16)] +
                               loc_v[pl.ds(g * 16, 16)] + pvec)

    pltpu.async_copy(block_v.at[pl.ds(v2 * _BR, _BR)],
                     hist_all_s.at[pl.ds(v2 * _NB + wid * _BR, _BR)], sem_h)
  for v2 in range(_NV):
    pltpu.make_async_copy(block_v.at[pl.ds(v2 * _BR, _BR)],
                          hist_all_s.at[pl.ds(v2 * _NB + wid * _BR, _BR)],
                          sem_h).wait()
  plsc.subcore_barrier()

  pltpu.async_copy(hist_all_s.at[pl.ds((2 * wid) * _NB, _NB)], hist_a, sem_h)
  pltpu.async_copy(hist_all_s.at[pl.ds((2 * wid + 1) * _NB, _NB)], hist_b,
                   sem_h)
  pltpu.make_async_copy(hist_all_s.at[pl.ds((2 * wid) * _NB, _NB)], hist_a,
                        sem_h).wait()
  pltpu.make_async_copy(hist_all_s.at[pl.ds((2 * wid) * _NB, _NB)], hist_b,
                        sem_h).wait()
  pltpu.make_async_copy(dst_hbm.at[pl.ds(off, _S)], vals_v.at[pl.ds(0, _S)],
                        sem_v).wait()

  mask14 = jnp.full((16,), (1 << 14) - 1, jnp.int32)

  def _chunk(c, pos_a, pos_b):
    for j in range(_CHUNK // 16):
      pa = keys_v[pl.ds(c * _CHUNK + j * 16, 16)]
      pb = keys_v[pl.ds(_SV + c * _CHUNK + j * 16, 16)]
      pos_a[pl.ds(j * 16, 16)] = (
          plsc.load_gather(hist_a, [pa & mask14]) + (pa >> 14))
      pos_b[pl.ds(j * 16, 16)] = (
          plsc.load_gather(hist_b, [pb & mask14]) + (pb >> 14))

  def _fire(c, pos_a, pos_b, sa, sb):
    pltpu.async_copy(vals_v.at[pl.ds(c * _CHUNK, _CHUNK)],
                     out_s.at[pos_a], sa)
    pltpu.async_copy(vals_v.at[pl.ds(_SV + c * _CHUNK, _CHUNK)],
                     out_s.at[pos_b], sb)

  def _drain(c, pos_a, pos_b, sa, sb):
    pltpu.make_async_copy(vals_v.at[pl.ds(c * _CHUNK, _CHUNK)],
                          out_s.at[pos_a], sa).wait()
    pltpu.make_async_copy(vals_v.at[pl.ds(_SV + c * _CHUNK, _CHUNK)],
                          out_s.at[pos_b], sb).wait()

  @pl.loop(0, _NCH // 2)
  def _(h):
    c0 = 2 * h
    c1 = 2 * h + 1

    @pl.when(h > 0)
    def _():
      _drain(c0 - 2, pos_a0, pos_b0, sem_a0, sem_b0)

    _chunk(c0, pos_a0, pos_b0)
    _fire(c0, pos_a0, pos_b0, sem_a0, sem_b0)

    @pl.when(h > 0)
    def _():
      _drain(c1 - 2, pos_a1, pos_b1, sem_a1, sem_b1)

    _chunk(c1, pos_a1, pos_b1)
    _fire(c1, pos_a1, pos_b1, sem_a1, sem_b1)

  _drain(_NCH - 2, pos_a0, pos_b0, sem_a0, sem_b0)
  _drain(_NCH - 1, pos_a1, pos_b1, sem_a1, sem_b1)

  iot = lax.iota(jnp.int32, 16)
  pa = keys_v[pl.ds(_NCH * _CHUNK, 16)]
  pb = keys_v[pl.ds(_SV + _NCH * _CHUNK, 16)]
  pos_a0[pl.ds(0, 16)] = (
      plsc.load_gather(hist_a, [pa & mask14]) + (pa >> 14))
  pos_b0[pl.ds(0, 16)] = (
      plsc.load_gather(hist_b, [pb & mask14]) + (pb >> 14))
  for j in range(1, _CHUNK // 16):
    pos_a0[pl.ds(j * 16, 16)] = _N + (2 * wid) * _PAD + (j - 1) * 16 + iot
    pos_b0[pl.ds(j * 16, 16)] = (_N + (2 * wid + 1) * _PAD +
                                 (j - 1) * 16 + iot)
  pltpu.sync_copy(vals_v.at[pl.ds(_NCH * _CHUNK, _CHUNK)], out_s.at[pos_a0])
  pltpu.sync_copy(vals_v.at[pl.ds(_SV + _NCH * _CHUNK, _CHUNK)],
                  out_s.at[pos_b0])
  plsc.subcore_barrier()

  pltpu.sync_copy(out_s.at[pl.ds(wid * _S, _S)], keys_v)
  pltpu.sync_copy(keys_v, out_hbm.at[pl.ds(wid * _S, _S)])


_sort = pl.kernel(
    _body,
    out_type=jax.ShapeDtypeStruct((_N,), jnp.int32),
    mesh=plsc.VectorSubcoreMesh(
        core_axis_name="c", subcore_axis_name="s", num_cores=1),
    compiler_params=pltpu.CompilerParams(needs_layout_passes=False),
    scratch_types=[
        pltpu.VMEM((_S,), jnp.int32),
        pltpu.VMEM((_S + _PAD,), jnp.int32),
        pltpu.VMEM((_NB,), jnp.int32),
        pltpu.VMEM((_NB,), jnp.int32),
        pltpu.VMEM((_NV * _BR,), jnp.int32),
        pltpu.VMEM((_BR,), jnp.int32),
        pltpu.VMEM((_BR,), jnp.int32),
        pltpu.VMEM((16,), jnp.int32),
        pltpu.VMEM((_CHUNK,), jnp.int32),
        pltpu.VMEM((_CHUNK,), jnp.int32),
        pltpu.VMEM((_CHUNK,), jnp.int32),
        pltpu.VMEM((_CHUNK,), jnp.int32),
        pltpu.SMEM((1,), jnp.int32),
        pltpu.VMEM_SHARED((_NV * _NB,), jnp.int32),
        pltpu.VMEM_SHARED((_NW * 16,), jnp.int32),
        pltpu.VMEM_SHARED((_OUT_S,), jnp.int32),
        pltpu.SemaphoreType.DMA,
        pltpu.SemaphoreType.DMA,
        pltpu.SemaphoreType.DMA,
        pltpu.SemaphoreType.DMA,
        pltpu.SemaphoreType.DMA,
        pltpu.SemaphoreType.DMA,
        pltpu.SemaphoreType.DMA,
    ],
)


@jax.jit
def kernel(edges, nodes):
  e = edges.astype(jnp.int32)
  flat = _sort(e[:, 0], e[:, 1])
  return flat.reshape(nodes.shape[0], -1)
